# baseline (device time: 58801 ns/iter reference)
import jax
import jax.numpy as jnp
from jax import lax
from jax.experimental import pallas as pl
from jax.experimental.pallas import tpu as pltpu

N_DEV = 16
N_FWD = 8
N_BWD = 7
G = 4


def kernel(x, w_mat):
    m_per, k = x.shape
    _, n_per = w_mat.shape
    m_g = m_per // G

    x = x.astype(jnp.bfloat16)
    w_mat = w_mat.astype(jnp.bfloat16)

    def body(x_ref, w_ref, out_ref, fwd_ref, bwd_ref,
             fwd_send_sems, fwd_recv_sems, bwd_send_sems, bwd_recv_sems):
        my = lax.axis_index("i")
        left = lax.rem(my + N_DEV - 1, N_DEV)
        right = lax.rem(my + 1, N_DEV)

        barrier_sem = pltpu.get_barrier_semaphore()
        for nbr in (left, right):
            pl.semaphore_signal(
                barrier_sem, inc=1,
                device_id=(nbr,), device_id_type=pl.DeviceIdType.MESH,
            )
        pl.semaphore_wait(barrier_sem, 2)

        def mk(ref, sems_s, sems_r, h, g, dev):
            rows = pl.ds(g * m_g, m_g)
            src = x_ref.at[rows, :] if h == 0 else ref.at[h - 1, rows, :]
            return pltpu.make_async_remote_copy(
                src_ref=src,
                dst_ref=ref.at[h, rows, :],
                send_sem=sems_s.at[h, g],
                recv_sem=sems_r.at[h, g],
                device_id=(dev,),
                device_id_type=pl.DeviceIdType.MESH,
            )

        def mk_fwd(h, g):
            return mk(fwd_ref, fwd_send_sems, fwd_recv_sems, h, g, right)

        def mk_bwd(h, g):
            return mk(bwd_ref, bwd_send_sems, bwd_recv_sems, h, g, left)

        fwd_rdmas = {}
        bwd_rdmas = {}
        for g in range(G):
            fwd_rdmas[(0, g)] = mk_fwd(0, g)
            fwd_rdmas[(0, g)].start()
            bwd_rdmas[(0, g)] = mk_bwd(0, g)
            bwd_rdmas[(0, g)].start()

        w = w_ref[...]

        def compute(chunk, origin):
            acc = jnp.dot(chunk, w, preferred_element_type=jnp.float32)
            out_ref[pl.ds(origin * m_per, m_per), :] = jnp.maximum(acc, 0.0)

        compute(x_ref[...], my)

        for h in range(N_FWD):
            for g in range(G):
                fwd_rdmas[(h, g)].wait_recv()
                if h + 1 < N_FWD:
                    r = mk_fwd(h + 1, g)
                    r.start()
                    fwd_rdmas[(h + 1, g)] = r
            if h < N_BWD:
                for g in range(G):
                    bwd_rdmas[(h, g)].wait_recv()
                    if h + 1 < N_BWD:
                        r = mk_bwd(h + 1, g)
                        r.start()
                        bwd_rdmas[(h + 1, g)] = r
            compute(fwd_ref[h], lax.rem(my + N_DEV - 1 - h, N_DEV))
            if h < N_BWD:
                compute(bwd_ref[h], lax.rem(my + 1 + h, N_DEV))

        for r in fwd_rdmas.values():
            r.wait_send()
        for r in bwd_rdmas.values():
            r.wait_send()

    return pl.pallas_call(
        body,
        out_shape=jax.ShapeDtypeStruct((k, n_per), jnp.float32),
        in_specs=[
            pl.BlockSpec(memory_space=pltpu.VMEM),
            pl.BlockSpec(memory_space=pltpu.VMEM),
        ],
        out_specs=pl.BlockSpec(memory_space=pltpu.VMEM),
        scratch_shapes=[
            pltpu.VMEM((N_FWD, m_per, k), jnp.bfloat16),
            pltpu.VMEM((N_BWD, m_per, k), jnp.bfloat16),
            pltpu.SemaphoreType.DMA((N_FWD, G)),
            pltpu.SemaphoreType.DMA((N_FWD, G)),
            pltpu.SemaphoreType.DMA((N_BWD, G)),
            pltpu.SemaphoreType.DMA((N_BWD, G)),
        ],
        compiler_params=pltpu.CompilerParams(collective_id=0),
    )(x, w_mat)


# device time: 56488 ns/iter; 1.0409x vs baseline; 1.0409x over previous
import jax
import jax.numpy as jnp
from jax import lax
from jax.experimental import pallas as pl
from jax.experimental.pallas import tpu as pltpu

N_DEV = 16
NZ = 4
NP = 4


def kernel(x, w_mat):
    m_per, k = x.shape
    _, n_per = w_mat.shape
    m_h = m_per // 2

    x = x.astype(jnp.bfloat16)
    w_mat = w_mat.astype(jnp.bfloat16)

    def body(x_ref, w_ref, out_ref, col_ref, pf_ref, pb_ref, hf_ref, hb_ref,
             col_ssem, col_rsem, af_ssem, af_rsem, ab_ssem, ab_rsem,
             bf_ssem, bf_rsem, bb_ssem, bb_rsem):
        my = lax.axis_index("i")
        p = lax.rem(my, NP)
        z = my // NP
        zb = my - p
        pnext = zb + lax.rem(p + 1, NP)
        pprev = zb + lax.rem(p + 3, NP)
        popp = zb + lax.rem(p + 2, NP)

        w = w_ref[...]

        def compute(chunk, origin):
            acc = jnp.dot(chunk, w, preferred_element_type=jnp.float32)
            out_ref[pl.ds(origin * m_per, m_per), :] = jnp.maximum(acc, 0.0)

        def compute_half(chunk, origin, half):
            acc = jnp.dot(chunk, w, preferred_element_type=jnp.float32)
            out_ref[pl.ds(origin * m_per + half * m_h, m_h), :] = (
                jnp.maximum(acc, 0.0))

        barrier_sem = pltpu.get_barrier_semaphore()
        for nbr in (pnext, pprev,
                    lax.rem(my + 4, N_DEV), lax.rem(my + 8, N_DEV),
                    lax.rem(my + 12, N_DEV)):
            pl.semaphore_signal(
                barrier_sem, inc=1,
                device_id=(nbr,), device_id_type=pl.DeviceIdType.MESH,
            )
        pl.semaphore_wait(barrier_sem, 5)

        col_sends = []
        for e in (1, 2, 3):
            c_up = e
            c_dn = 4 - e

            @pl.when(z + e <= NZ - 1)
            def _(e=e, c_up=c_up):
                r = pltpu.make_async_remote_copy(
                    src_ref=x_ref,
                    dst_ref=col_ref.at[c_up - 1],
                    send_sem=col_ssem.at[e - 1],
                    recv_sem=col_rsem.at[c_up - 1],
                    device_id=(my + 4 * e,),
                    device_id_type=pl.DeviceIdType.MESH,
                )
                r.start()

            @pl.when(z - e >= 0)
            def _(e=e, c_dn=c_dn):
                r = pltpu.make_async_remote_copy(
                    src_ref=x_ref,
                    dst_ref=col_ref.at[c_dn - 1],
                    send_sem=col_ssem.at[3 + e - 1],
                    recv_sem=col_rsem.at[c_dn - 1],
                    device_id=(my - 4 * e,),
                    device_id_type=pl.DeviceIdType.MESH,
                )
                r.start()
            col_sends.append(e)

        def mk_a(c, dev, dst, ssem, rsem):
            src = x_ref if c == 0 else col_ref.at[c - 1]
            return pltpu.make_async_remote_copy(
                src_ref=src, dst_ref=dst.at[c],
                send_sem=ssem.at[c], recv_sem=rsem.at[c],
                device_id=(dev,), device_id_type=pl.DeviceIdType.MESH,
            )

        def mk_af(c):
            return mk_a(c, pnext, pf_ref, af_ssem, af_rsem)

        def mk_ab(c):
            return mk_a(c, pprev, pb_ref, ab_ssem, ab_rsem)

        def mk_bf(c):
            return pltpu.make_async_remote_copy(
                src_ref=pf_ref.at[c, pl.ds(0, m_h), :],
                dst_ref=hf_ref.at[c],
                send_sem=bf_ssem.at[c], recv_sem=bf_rsem.at[c],
                device_id=(pnext,), device_id_type=pl.DeviceIdType.MESH,
            )

        def mk_bb(c):
            return pltpu.make_async_remote_copy(
                src_ref=pb_ref.at[c, pl.ds(m_h, m_h), :],
                dst_ref=hb_ref.at[c],
                send_sem=bb_ssem.at[c], recv_sem=bb_rsem.at[c],
                device_id=(pprev,), device_id_type=pl.DeviceIdType.MESH,
            )

        a_rdmas = {}
        a_rdmas[("f", 0)] = mk_af(0)
        a_rdmas[("f", 0)].start()
        a_rdmas[("b", 0)] = mk_ab(0)
        a_rdmas[("b", 0)].start()
        compute(x_ref[...], my)

        for e in (1, 2, 3):
            for updown, c in ((0, e), (1, 4 - e)):
                cond = (z - e >= 0) if updown == 0 else (z + e <= NZ - 1)
                origin = (lax.rem(my + N_DEV - 4 * e, N_DEV) if updown == 0
                          else lax.rem(my + 4 * e, N_DEV))

                @pl.when(cond)
                def _(c=c, origin=origin):
                    wait = pltpu.make_async_remote_copy(
                        src_ref=x_ref, dst_ref=col_ref.at[c - 1],
                        send_sem=col_ssem.at[0], recv_sem=col_rsem.at[c - 1],
                        device_id=(my,), device_id_type=pl.DeviceIdType.MESH,
                    )
                    wait.wait_recv()
                    af = mk_af(c)
                    af.start()
                    ab = mk_ab(c)
                    ab.start()
                    compute(col_ref[c - 1], origin)
                a_rdmas[("f", c)] = mk_af(c)
                a_rdmas[("b", c)] = mk_ab(c)

        b_rdmas = {}
        for c in range(4):
            a_rdmas[("f", c)].wait_recv()
            r = mk_bf(c)
            r.start()
            b_rdmas[("f", c)] = r
            compute(pf_ref[c], lax.rem(pprev + N_DEV - 4 * c, N_DEV))

            a_rdmas[("b", c)].wait_recv()
            r = mk_bb(c)
            r.start()
            b_rdmas[("b", c)] = r
            compute(pb_ref[c], lax.rem(pnext + N_DEV - 4 * c, N_DEV))

        for c in range(4):
            o = lax.rem(popp + N_DEV - 4 * c, N_DEV)
            b_rdmas[("f", c)].wait_recv()
            compute_half(hf_ref[c], o, 0)
            b_rdmas[("b", c)].wait_recv()
            compute_half(hb_ref[c], o, 1)

        for e in (1, 2, 3):
            @pl.when(z + e <= NZ - 1)
            def _(e=e):
                pltpu.make_async_remote_copy(
                    src_ref=x_ref, dst_ref=col_ref.at[0],
                    send_sem=col_ssem.at[e - 1], recv_sem=col_rsem.at[0],
                    device_id=(my,), device_id_type=pl.DeviceIdType.MESH,
                ).wait_send()

            @pl.when(z - e >= 0)
            def _(e=e):
                pltpu.make_async_remote_copy(
                    src_ref=x_ref, dst_ref=col_ref.at[0],
                    send_sem=col_ssem.at[3 + e - 1], recv_sem=col_rsem.at[0],
                    device_id=(my,), device_id_type=pl.DeviceIdType.MESH,
                ).wait_send()
        for r in a_rdmas.values():
            r.wait_send()
        for r in b_rdmas.values():
            r.wait_send()

    return pl.pallas_call(
        body,
        out_shape=jax.ShapeDtypeStruct((k, n_per), jnp.float32),
        in_specs=[
            pl.BlockSpec(memory_space=pltpu.VMEM),
            pl.BlockSpec(memory_space=pltpu.VMEM),
        ],
        out_specs=pl.BlockSpec(memory_space=pltpu.VMEM),
        scratch_shapes=[
            pltpu.VMEM((3, m_per, k), jnp.bfloat16),
            pltpu.VMEM((4, m_per, k), jnp.bfloat16),
            pltpu.VMEM((4, m_per, k), jnp.bfloat16),
            pltpu.VMEM((4, m_h, k), jnp.bfloat16),
            pltpu.VMEM((4, m_h, k), jnp.bfloat16),
            pltpu.SemaphoreType.DMA((6,)),
            pltpu.SemaphoreType.DMA((3,)),
            pltpu.SemaphoreType.DMA((4,)),
            pltpu.SemaphoreType.DMA((4,)),
            pltpu.SemaphoreType.DMA((4,)),
            pltpu.SemaphoreType.DMA((4,)),
            pltpu.SemaphoreType.DMA((4,)),
            pltpu.SemaphoreType.DMA((4,)),
            pltpu.SemaphoreType.DMA((4,)),
            pltpu.SemaphoreType.DMA((4,)),
        ],
        compiler_params=pltpu.CompilerParams(collective_id=0),
    )(x, w_mat)


# device time: 55389 ns/iter; 1.0616x vs baseline; 1.0198x over previous
import jax
import jax.numpy as jnp
from jax import lax
from jax.experimental import pallas as pl
from jax.experimental.pallas import tpu as pltpu

N_DEV = 16
NZ = 4
NP = 4


def kernel(x, w_mat):
    m_per, k = x.shape
    _, n_per = w_mat.shape
    m_h = m_per // 2

    x = x.astype(jnp.bfloat16)
    w_mat = w_mat.astype(jnp.bfloat16)

    def body(x_ref, w_ref, out_ref, col_ref, pf_ref, pb_ref, hf_ref, hb_ref,
             col_ssem, col_rsem, af_ssem, af_rsem, ab_ssem, ab_rsem,
             bf_ssem, bf_rsem, bb_ssem, bb_rsem):
        my = lax.axis_index("i")
        p = lax.rem(my, NP)
        z = my // NP
        zb = my - p
        pnext = zb + lax.rem(p + 1, NP)
        pprev = zb + lax.rem(p + 3, NP)
        popp = zb + lax.rem(p + 2, NP)

        w = w_ref[...]

        def compute(chunk, origin):
            acc = jnp.dot(chunk, w, preferred_element_type=jnp.float32)
            out_ref[pl.ds(origin * m_per, m_per), :] = jnp.maximum(acc, 0.0)

        def compute_half(chunk, origin, half):
            acc = jnp.dot(chunk, w, preferred_element_type=jnp.float32)
            out_ref[pl.ds(origin * m_per + half * m_h, m_h), :] = (
                jnp.maximum(acc, 0.0))

        barrier_sem = pltpu.get_barrier_semaphore()
        for nbr in (pnext, pprev,
                    lax.rem(my + 4, N_DEV), lax.rem(my + 8, N_DEV),
                    lax.rem(my + 12, N_DEV)):
            pl.semaphore_signal(
                barrier_sem, inc=1,
                device_id=(nbr,), device_id_type=pl.DeviceIdType.MESH,
            )
        pl.semaphore_wait(barrier_sem, 5)

        col_sends = []
        for e in (1, 2, 3):
            c_up = e
            c_dn = 4 - e

            @pl.when(z + e <= NZ - 1)
            def _(e=e, c_up=c_up):
                r = pltpu.make_async_remote_copy(
                    src_ref=x_ref,
                    dst_ref=col_ref.at[c_up - 1],
                    send_sem=col_ssem.at[e - 1],
                    recv_sem=col_rsem.at[c_up - 1],
                    device_id=(my + 4 * e,),
                    device_id_type=pl.DeviceIdType.MESH,
                )
                r.start()

            @pl.when(z - e >= 0)
            def _(e=e, c_dn=c_dn):
                r = pltpu.make_async_remote_copy(
                    src_ref=x_ref,
                    dst_ref=col_ref.at[c_dn - 1],
                    send_sem=col_ssem.at[3 + e - 1],
                    recv_sem=col_rsem.at[c_dn - 1],
                    device_id=(my - 4 * e,),
                    device_id_type=pl.DeviceIdType.MESH,
                )
                r.start()
            col_sends.append(e)

        def mk_a(c, half, dev, dst, ssem, rsem):
            rows = pl.ds(half * m_h, m_h)
            src = x_ref.at[rows, :] if c == 0 else col_ref.at[c - 1, rows, :]
            return pltpu.make_async_remote_copy(
                src_ref=src, dst_ref=dst.at[c, rows, :],
                send_sem=ssem.at[c, half], recv_sem=rsem.at[c, half],
                device_id=(dev,), device_id_type=pl.DeviceIdType.MESH,
            )

        def mk_af(c, half):
            return mk_a(c, half, pnext, pf_ref, af_ssem, af_rsem)

        def mk_ab(c, half):
            return mk_a(c, half, pprev, pb_ref, ab_ssem, ab_rsem)

        def mk_bf(c):
            return pltpu.make_async_remote_copy(
                src_ref=pf_ref.at[c, pl.ds(0, m_h), :],
                dst_ref=hf_ref.at[c],
                send_sem=bf_ssem.at[c], recv_sem=bf_rsem.at[c],
                device_id=(pnext,), device_id_type=pl.DeviceIdType.MESH,
            )

        def mk_bb(c):
            return pltpu.make_async_remote_copy(
                src_ref=pb_ref.at[c, pl.ds(m_h, m_h), :],
                dst_ref=hb_ref.at[c],
                send_sem=bb_ssem.at[c], recv_sem=bb_rsem.at[c],
                device_id=(pprev,), device_id_type=pl.DeviceIdType.MESH,
            )

        a_rdmas = {}

        def start_a(c):
            for key, h in ((("f", c, 0), 0), (("f", c, 1), 1)):
                a_rdmas[key] = mk_af(c, h)
                a_rdmas[key].start()
            for key, h in ((("b", c, 1), 1), (("b", c, 0), 0)):
                a_rdmas[key] = mk_ab(c, h)
                a_rdmas[key].start()

        start_a(0)
        compute(x_ref[...], my)

        for e in (1, 2, 3):
            for updown, c in ((0, e), (1, 4 - e)):
                cond = (z - e >= 0) if updown == 0 else (z + e <= NZ - 1)
                origin = (lax.rem(my + N_DEV - 4 * e, N_DEV) if updown == 0
                          else lax.rem(my + 4 * e, N_DEV))

                @pl.when(cond)
                def _(c=c, origin=origin):
                    wait = pltpu.make_async_remote_copy(
                        src_ref=x_ref, dst_ref=col_ref.at[c - 1],
                        send_sem=col_ssem.at[0], recv_sem=col_rsem.at[c - 1],
                        device_id=(my,), device_id_type=pl.DeviceIdType.MESH,
                    )
                    wait.wait_recv()
                    for h in (0, 1):
                        mk_af(c, h).start()
                    for h in (1, 0):
                        mk_ab(c, h).start()
                    compute(col_ref[c - 1], origin)
                for h in (0, 1):
                    a_rdmas[("f", c, h)] = mk_af(c, h)
                    a_rdmas[("b", c, h)] = mk_ab(c, h)

        b_rdmas = {}
        for c in range(4):
            a_rdmas[("f", c, 0)].wait_recv()
            r = mk_bf(c)
            r.start()
            b_rdmas[("f", c)] = r
            a_rdmas[("f", c, 1)].wait_recv()
            compute(pf_ref[c], lax.rem(pprev + N_DEV - 4 * c, N_DEV))

            a_rdmas[("b", c, 1)].wait_recv()
            r = mk_bb(c)
            r.start()
            b_rdmas[("b", c)] = r
            a_rdmas[("b", c, 0)].wait_recv()
            compute(pb_ref[c], lax.rem(pnext + N_DEV - 4 * c, N_DEV))

        for c in range(4):
            o = lax.rem(popp + N_DEV - 4 * c, N_DEV)
            b_rdmas[("f", c)].wait_recv()
            compute_half(hf_ref[c], o, 0)
            b_rdmas[("b", c)].wait_recv()
            compute_half(hb_ref[c], o, 1)

        for e in (1, 2, 3):
            @pl.when(z + e <= NZ - 1)
            def _(e=e):
                pltpu.make_async_remote_copy(
                    src_ref=x_ref, dst_ref=col_ref.at[0],
                    send_sem=col_ssem.at[e - 1], recv_sem=col_rsem.at[0],
                    device_id=(my,), device_id_type=pl.DeviceIdType.MESH,
                ).wait_send()

            @pl.when(z - e >= 0)
            def _(e=e):
                pltpu.make_async_remote_copy(
                    src_ref=x_ref, dst_ref=col_ref.at[0],
                    send_sem=col_ssem.at[3 + e - 1], recv_sem=col_rsem.at[0],
                    device_id=(my,), device_id_type=pl.DeviceIdType.MESH,
                ).wait_send()
        for r in a_rdmas.values():
            r.wait_send()
        for r in b_rdmas.values():
            r.wait_send()

    return pl.pallas_call(
        body,
        out_shape=jax.ShapeDtypeStruct((k, n_per), jnp.float32),
        in_specs=[
            pl.BlockSpec(memory_space=pltpu.VMEM),
            pl.BlockSpec(memory_space=pltpu.VMEM),
        ],
        out_specs=pl.BlockSpec(memory_space=pltpu.VMEM),
        scratch_shapes=[
            pltpu.VMEM((3, m_per, k), jnp.bfloat16),
            pltpu.VMEM((4, m_per, k), jnp.bfloat16),
            pltpu.VMEM((4, m_per, k), jnp.bfloat16),
            pltpu.VMEM((4, m_h, k), jnp.bfloat16),
            pltpu.VMEM((4, m_h, k), jnp.bfloat16),
            pltpu.SemaphoreType.DMA((6,)),
            pltpu.SemaphoreType.DMA((3,)),
            pltpu.SemaphoreType.DMA((4, 2)),
            pltpu.SemaphoreType.DMA((4, 2)),
            pltpu.SemaphoreType.DMA((4, 2)),
            pltpu.SemaphoreType.DMA((4, 2)),
            pltpu.SemaphoreType.DMA((4,)),
            pltpu.SemaphoreType.DMA((4,)),
            pltpu.SemaphoreType.DMA((4,)),
            pltpu.SemaphoreType.DMA((4,)),
        ],
        compiler_params=pltpu.CompilerParams(collective_id=0),
    )(x, w_mat)
